# Initial kernel scaffold; baseline (speedup 1.0000x reference)
import jax
import jax.numpy as jnp
from jax.experimental import pallas as pl


def kernel(weight, steps):
    raise NotImplementedError("write your pallas kernel here")


# speedup vs baseline: 9.8155x; 9.8155x over previous
import functools

import jax
import jax.numpy as jnp
from jax import lax
from jax.experimental import pallas as pl
from jax.experimental.pallas import tpu as pltpu

_INPUT_SIZE = 1024
_START = 40000
_END = 100000
_DENS = (0.5, 0.5, 1.0)
_NROWS = 6144
_NCOLS = 1024
_BR = 8
_BC = 4
_SR = 256
_SC = 256
_CHUNK = 256
_NCHUNK = _NROWS // _CHUNK


def _scores_body(w_ref, p_ref, s_ref):
    x = w_ref[...]
    x2 = x * x
    y = jnp.sum(x2.reshape(_CHUNK // _BR, _BR, _NCOLS), axis=1)
    s_ref[...] = jax.lax.dot_general(
        y, p_ref[...], (((1,), (0,)), ((), ())),
        precision=jax.lax.Precision.HIGHEST,
        preferred_element_type=jnp.float32)


def _select_body(s_ref, idx_ref, m_ref):
    for g in range(3):
        sg = s_ref[g * _SR:(g + 1) * _SR, :]
        u = jax.lax.bitcast_convert_type(sg, jnp.int32)
        k = idx_ref[g]

        def body(i, x):
            cand = x | (1 << (30 - i))
            cnt = jnp.sum((u < cand).astype(jnp.int32))
            return jnp.where(cnt <= k, cand, x)

        t = lax.fori_loop(0, 31, body, jnp.int32(0))
        m_ref[g * _SR:(g + 1) * _SR, :] = (u >= t).astype(jnp.float32)


def _expand_body(m_ref, e_ref, o_ref):
    m = m_ref[...]
    mr = jnp.broadcast_to(m[:, None, :], (_CHUNK // _BR, _BR, _SC))
    mr = mr.reshape(_CHUNK, _SC)
    o_ref[...] = jax.lax.dot_general(
        mr, e_ref[...], (((1,), (0,)), ((), ())),
        preferred_element_type=jnp.float32)


def kernel(weight, steps):
    dens = []
    for k in range(3):
        r = 1.0 - (steps - _START) / (_END - _START)
        dens.append(jnp.where(steps < _END,
                              1.0 - (1.0 - _DENS[k]) * (1.0 - r ** 3),
                              _DENS[k]))
    nblk = _SR * _SC
    idx = jnp.stack([jnp.round(nblk * (1.0 - d)).astype(jnp.int32)
                     for d in dens])

    ci = jax.lax.broadcasted_iota(jnp.int32, (_NCOLS, _SC), 0)
    cj = jax.lax.broadcasted_iota(jnp.int32, (_NCOLS, _SC), 1)
    p = (ci // _BC == cj).astype(jnp.float32)
    e = p.T

    scores = pl.pallas_call(
        _scores_body,
        grid=(_NCHUNK,),
        in_specs=[
            pl.BlockSpec((_CHUNK, _NCOLS), lambda i: (i, 0)),
            pl.BlockSpec((_NCOLS, _SC), lambda i: (0, 0)),
        ],
        out_specs=pl.BlockSpec((_CHUNK // _BR, _SC), lambda i: (i, 0)),
        out_shape=jax.ShapeDtypeStruct((3 * _SR, _SC), jnp.float32),
    )(weight, p)

    cmask = pl.pallas_call(
        _select_body,
        in_specs=[
            pl.BlockSpec(memory_space=pltpu.VMEM),
            pl.BlockSpec(memory_space=pltpu.SMEM),
        ],
        out_specs=pl.BlockSpec(memory_space=pltpu.VMEM),
        out_shape=jax.ShapeDtypeStruct((3 * _SR, _SC), jnp.float32),
    )(scores, idx)

    out = pl.pallas_call(
        _expand_body,
        grid=(_NCHUNK,),
        in_specs=[
            pl.BlockSpec((_CHUNK // _BR, _SC), lambda i: (i, 0)),
            pl.BlockSpec((_SC, _NCOLS), lambda i: (0, 0)),
        ],
        out_specs=pl.BlockSpec((_CHUNK, _NCOLS), lambda i: (i, 0)),
        out_shape=jax.ShapeDtypeStruct((_NROWS, _NCOLS), jnp.float32),
    )(cmask, e)
    return out


# speedup vs baseline: 11.8874x; 1.2111x over previous
import jax
import jax.numpy as jnp
from jax import lax
from jax.experimental import pallas as pl
from jax.experimental.pallas import tpu as pltpu

_INPUT_SIZE = 1024
_START = 40000
_END = 100000
_DENS = (0.5, 0.5, 1.0)
_NROWS = 6144
_NCOLS = 1024
_BR = 8
_BC = 4
_SR = 256
_SC = 256
_CHUNK = 256
_NCHUNK = _NROWS // _CHUNK
_CPG = _NCHUNK // 3


def _scores_body(w_ref, p_ref, s_ref):
    x = w_ref[...]
    x2 = x * x
    y = jnp.sum(x2.reshape(_CHUNK // _BR, _BR, _NCOLS), axis=1)
    s_ref[...] = jax.lax.dot_general(
        y, p_ref[...], (((1,), (0,)), ((), ())),
        precision=jax.lax.Precision.HIGHEST,
        preferred_element_type=jnp.float32)


def _mask_body(s_ref, idx_ref, e_ref, o_ref, ts_ref):
    i = pl.program_id(0)

    @pl.when(i == 0)
    def _select():
        u3 = jnp.stack([
            jax.lax.bitcast_convert_type(s_ref[g * _SR:(g + 1) * _SR, :],
                                         jnp.int32)
            for g in range(3)])
        kvec = jnp.stack([idx_ref[0], idx_ref[1], idx_ref[2]])

        def body(it, x):
            cand = x | (jnp.int32(1) << (30 - it))
            cnt = jnp.sum((u3 < cand[:, None, None]).astype(jnp.int32),
                          axis=(1, 2))
            return jnp.where(cnt <= kvec, cand, x)

        x = lax.fori_loop(0, 31, body, jnp.zeros((3,), jnp.int32))
        for g in range(3):
            ts_ref[g] = x[g]

    @pl.when(i > 0)
    def _expand():
        c = i - 1
        t = ts_ref[c // _CPG]
        s = s_ref[pl.ds(c * (_CHUNK // _BR), _CHUNK // _BR), :]
        m = (jax.lax.bitcast_convert_type(s, jnp.int32) >= t
             ).astype(jnp.float32)
        mr = jnp.broadcast_to(m[:, None, :], (_CHUNK // _BR, _BR, _SC))
        mr = mr.reshape(_CHUNK, _SC)
        o_ref[...] = jax.lax.dot_general(
            mr, e_ref[...], (((1,), (0,)), ((), ())),
            preferred_element_type=jnp.float32)


def kernel(weight, steps):
    dens = []
    for k in range(3):
        r = 1.0 - (steps - _START) / (_END - _START)
        dens.append(jnp.where(steps < _END,
                              1.0 - (1.0 - _DENS[k]) * (1.0 - r ** 3),
                              _DENS[k]))
    nblk = _SR * _SC
    idx = jnp.stack([jnp.round(nblk * (1.0 - d)).astype(jnp.int32)
                     for d in dens])

    ci = jax.lax.broadcasted_iota(jnp.int32, (_NCOLS, _SC), 0)
    cj = jax.lax.broadcasted_iota(jnp.int32, (_NCOLS, _SC), 1)
    p = (ci // _BC == cj).astype(jnp.float32)
    e = p.T

    scores = pl.pallas_call(
        _scores_body,
        grid=(_NCHUNK,),
        in_specs=[
            pl.BlockSpec((_CHUNK, _NCOLS), lambda i: (i, 0)),
            pl.BlockSpec((_NCOLS, _SC), lambda i: (0, 0)),
        ],
        out_specs=pl.BlockSpec((_CHUNK // _BR, _SC), lambda i: (i, 0)),
        out_shape=jax.ShapeDtypeStruct((3 * _SR, _SC), jnp.float32),
    )(weight, p)

    out = pl.pallas_call(
        _mask_body,
        grid=(_NCHUNK + 1,),
        in_specs=[
            pl.BlockSpec(memory_space=pltpu.VMEM),
            pl.BlockSpec(memory_space=pltpu.SMEM),
            pl.BlockSpec((_SC, _NCOLS), lambda i: (0, 0)),
        ],
        out_specs=pl.BlockSpec((_CHUNK, _NCOLS),
                               lambda i: (jnp.maximum(i - 1, 0), 0)),
        out_shape=jax.ShapeDtypeStruct((_NROWS, _NCOLS), jnp.float32),
        scratch_shapes=[pltpu.SMEM((3,), jnp.int32)],
    )(scores, idx, e)
    return out


# speedup vs baseline: 13.3776x; 1.1254x over previous
import jax
import jax.numpy as jnp
from jax import lax
from jax.experimental import pallas as pl
from jax.experimental.pallas import tpu as pltpu

_START = 40000
_END = 100000
_DENS = (0.5, 0.5, 1.0)
_NROWS = 6144
_NCOLS = 1024
_BR = 8
_BC = 4
_SR = 256
_SC = 256
_CHUNK = 256
_NCHUNK = _NROWS // _CHUNK
_CPG = _NCHUNK // 3
_SRC = _CHUNK // _BR


def _scores_body(w_ref, s_ref):
    x = w_ref[...]
    y = jnp.sum((x * x).reshape(_SRC, _BR, _NCOLS), axis=1)
    yt = y.T
    st = jnp.sum(yt.reshape(_SC, _BC, _SRC), axis=1)
    s_ref[...] = st.T


def _mask_body(s_ref, idx_ref, o_ref, ts_ref):
    i = pl.program_id(0)

    @pl.when(i == 0)
    def _select():
        u3 = jnp.stack([
            jax.lax.bitcast_convert_type(s_ref[g * _SR:(g + 1) * _SR, :],
                                         jnp.int32)
            for g in range(3)])
        kvec = jnp.stack([idx_ref[0], idx_ref[1], idx_ref[2]])

        def body(it, x):
            cand = x | (jnp.int32(1) << (30 - it))
            cnt = jnp.sum((u3 < cand[:, None, None]).astype(jnp.int32),
                          axis=(1, 2))
            return jnp.where(cnt <= kvec, cand, x)

        x = lax.fori_loop(0, 31, body, jnp.zeros((3,), jnp.int32))
        for g in range(3):
            ts_ref[g] = x[g]

    @pl.when(i > 0)
    def _expand():
        c = i - 1
        t = ts_ref[c // _CPG]
        s = s_ref[pl.ds(c * _SRC, _SRC), :]
        m = (jax.lax.bitcast_convert_type(s, jnp.int32) >= t
             ).astype(jnp.float32)
        mt = m.T
        m4 = jnp.broadcast_to(mt[:, None, :], (_SC, _BC, _SRC))
        m4 = m4.reshape(_NCOLS, _SRC)
        m4t = m4.T
        mr = jnp.broadcast_to(m4t[:, None, :], (_SRC, _BR, _NCOLS))
        o_ref[...] = mr.reshape(_CHUNK, _NCOLS)


def kernel(weight, steps):
    dens = []
    for k in range(3):
        r = 1.0 - (steps - _START) / (_END - _START)
        dens.append(jnp.where(steps < _END,
                              1.0 - (1.0 - _DENS[k]) * (1.0 - r ** 3),
                              _DENS[k]))
    nblk = _SR * _SC
    idx = jnp.stack([jnp.round(nblk * (1.0 - d)).astype(jnp.int32)
                     for d in dens])

    scores = pl.pallas_call(
        _scores_body,
        grid=(_NCHUNK,),
        in_specs=[pl.BlockSpec((_CHUNK, _NCOLS), lambda i: (i, 0))],
        out_specs=pl.BlockSpec((_SRC, _SC), lambda i: (i, 0)),
        out_shape=jax.ShapeDtypeStruct((3 * _SR, _SC), jnp.float32),
    )(weight)

    out = pl.pallas_call(
        _mask_body,
        grid=(_NCHUNK + 1,),
        in_specs=[
            pl.BlockSpec(memory_space=pltpu.VMEM),
            pl.BlockSpec(memory_space=pltpu.SMEM),
        ],
        out_specs=pl.BlockSpec((_CHUNK, _NCOLS),
                               lambda i: (jnp.maximum(i - 1, 0), 0)),
        out_shape=jax.ShapeDtypeStruct((_NROWS, _NCOLS), jnp.float32),
        scratch_shapes=[pltpu.SMEM((3,), jnp.int32)],
    )(scores, idx)
    return out


# speedup vs baseline: 20.9356x; 1.5650x over previous
import jax
import jax.numpy as jnp
from jax import lax
from jax.experimental import pallas as pl
from jax.experimental.pallas import tpu as pltpu

_START = 40000
_END = 100000
_DENS = (0.5, 0.5, 1.0)
_NROWS = 6144
_NCOLS = 1024
_BR = 8
_BC = 4
_SR = 256
_SC = 256
_CHUNK = 1024
_NCHUNK = _NROWS // _CHUNK
_CPG = _NCHUNK // 3
_SRC = _CHUNK // _BR


def _scores_body(w_ref, s_ref):
    x = w_ref[...]
    y = jnp.sum((x * x).reshape(_SRC, _BR, _NCOLS), axis=1)
    yt = y.T
    st = jnp.sum(yt.reshape(_SC, _BC, _SRC), axis=1)
    s_ref[...] = st.T


def _mask_body(s_ref, idx_ref, o_ref, ts_ref):
    i = pl.program_id(0)

    @pl.when(i == 0)
    def _select():
        u3 = jnp.stack([
            jax.lax.bitcast_convert_type(s_ref[g * _SR:(g + 1) * _SR, :],
                                         jnp.int32)
            for g in range(3)])
        kvec = jnp.stack([idx_ref[0], idx_ref[1], idx_ref[2]])

        def body(it, x):
            cand = x | (jnp.int32(1) << (30 - it))
            cnt = jnp.sum((u3 < cand[:, None, None]).astype(jnp.int32),
                          axis=(1, 2))
            return jnp.where(cnt <= kvec, cand, x)

        x = lax.fori_loop(0, 31, body, jnp.zeros((3,), jnp.int32))
        for g in range(3):
            ts_ref[g] = x[g]

    @pl.when(i > 0)
    def _expand():
        c = i - 1
        t = ts_ref[c // _CPG]
        s = s_ref[pl.ds(c * _SRC, _SRC), :]
        m = (jax.lax.bitcast_convert_type(s, jnp.int32) >= t
             ).astype(jnp.float32)
        mt = m.T
        m4 = jnp.broadcast_to(mt[:, None, :], (_SC, _BC, _SRC))
        m4 = m4.reshape(_NCOLS, _SRC)
        m4t = m4.T
        mr = jnp.broadcast_to(m4t[:, None, :], (_SRC, _BR, _NCOLS))
        o_ref[...] = mr.reshape(_CHUNK, _NCOLS)


def kernel(weight, steps):
    dens = []
    for k in range(3):
        r = 1.0 - (steps - _START) / (_END - _START)
        dens.append(jnp.where(steps < _END,
                              1.0 - (1.0 - _DENS[k]) * (1.0 - r ** 3),
                              _DENS[k]))
    nblk = _SR * _SC
    idx = jnp.stack([jnp.round(nblk * (1.0 - d)).astype(jnp.int32)
                     for d in dens])

    scores = pl.pallas_call(
        _scores_body,
        grid=(_NCHUNK,),
        in_specs=[pl.BlockSpec((_CHUNK, _NCOLS), lambda i: (i, 0))],
        out_specs=pl.BlockSpec((_SRC, _SC), lambda i: (i, 0)),
        out_shape=jax.ShapeDtypeStruct((3 * _SR, _SC), jnp.float32),
    )(weight)

    out = pl.pallas_call(
        _mask_body,
        grid=(_NCHUNK + 1,),
        in_specs=[
            pl.BlockSpec(memory_space=pltpu.VMEM),
            pl.BlockSpec(memory_space=pltpu.SMEM),
        ],
        out_specs=pl.BlockSpec((_CHUNK, _NCOLS),
                               lambda i: (jnp.maximum(i - 1, 0), 0)),
        out_shape=jax.ShapeDtypeStruct((_NROWS, _NCOLS), jnp.float32),
        scratch_shapes=[pltpu.SMEM((3,), jnp.int32)],
    )(scores, idx)
    return out
